# baseline (speedup 1.0000x reference)
import jax
import jax.numpy as jnp
from jax import lax
from jax.experimental import pallas as pl
from jax.experimental.pallas import tpu as pltpu
from jax.experimental.pallas import tpu_sc as plsc

N = 10000
D = 128
DH = 64
E = 320000
NP = 10240
NC = 2
NS = 16
CH = 128
NH = 2
CPH = 80
NCHUNK = NH * CPH
EPT_P = NCHUNK * CH
EPT = E // NS
RPT = NP // NS
CB = 64
RSUB = RPT // CB

_A = 1.0
_B = 1.0
_LLO = -1.0
_RHI = 1.0


def _coef_vector(alphas):
    al = jnp.tanh(alphas)
    a, b, l, r = _A, _B, _LLO, _RHI
    coef1 = (a - b) / 2 - (a + b + 2) / 2 * (l + r) / (r - l)
    coef2 = (a + b + 2) / (r - l)
    cs = [coef1 * al[0], coef2 * al[0]]
    for L in (2, 3):
        coef_l = 2 * L * (L + a + b) * (2 * L - 2 + a + b)
        coef_lm1_1 = (2 * L + a + b - 1) * (2 * L + a + b) * (2 * L + a + b - 2)
        coef_lm1_2 = (2 * L + a + b - 1) * (a ** 2 - b ** 2)
        coef_lm2 = 2 * (L - 1 + a) * (L - 1 + b) * (2 * L + a + b)
        tmp1 = al[L - 1] * (coef_lm1_1 / coef_l)
        tmp2 = al[L - 1] * (coef_lm1_2 / coef_l)
        tmp3 = al[L - 1] * al[L - 2] * (coef_lm2 / coef_l)
        tmp1_2 = tmp1 * (2 / (r - l))
        tmp2_2 = tmp1 * ((r + l) / (r - l)) + tmp2
        cs += [tmp1_2, tmp2_2, tmp3]
    cs += [jnp.float32(0.0)] * 8
    return jnp.stack([jnp.float32(c) for c in cs])


def _rsqrt16(d):
    s = d
    for _ in range(16):
        s = 0.5 * (s + d / s)
    return 1.0 / s


def _sc_body(xs_ref, row_ref, col_ref, w_ref, coef_ref,
             y1_ref, y2_ref, y3_ref,
             acc, accd, dinv_sh,
             row_h, colp_h, val_v, dinv_v,
             gbuf0, gbuf1, p2buf, ones1, dvec, coef_v,
             gsem0, gsem1, ssem0, ssem1):
    c = lax.axis_index("c")
    s = lax.axis_index("s")
    off = c * NP
    rbase = s * RPT

    pltpu.sync_copy(coef_ref, coef_v)
    zv = jnp.zeros((16,), jnp.float32)
    onev = jnp.ones((16,), jnp.float32)
    for g in range(CH // 16):
        ones1[pl.ds(g * 16, 16)] = onev
    for k in range(RPT // 16):
        dvec[pl.ds(k * 16, 16)] = zv

    pltpu.sync_copy(dvec, accd.at[pl.ds(rbase, RPT)])
    plsc.subcore_barrier()
    for h in range(NH):
        pltpu.sync_copy(row_ref.at[s, h], row_h)

        def _deg(j, _):
            pltpu.sync_copy(ones1, accd.at[row_h.at[j]], add=True)
            return 0
        lax.fori_loop(0, CPH, _deg, 0)
    plsc.subcore_barrier()

    pltpu.sync_copy(accd.at[pl.ds(rbase, RPT)], dvec)

    def _dx(k, _):
        d = dvec[pl.ds(k * 16, 16)]
        d = jnp.where(d < 0.5, d + 1.0, d)
        dvec[pl.ds(k * 16, 16)] = _rsqrt16(d)
        return 0
    lax.fori_loop(0, RPT // 16, _dx, 0)
    pltpu.sync_copy(dvec, dinv_sh.at[pl.ds(rbase, RPT)])
    plsc.subcore_barrier()
    pltpu.sync_copy(dinv_sh, dinv_v)

    for h in range(NH):
        pltpu.sync_copy(row_ref.at[s, h], row_h)
        pltpu.sync_copy(col_ref.at[s, h], colp_h)
        pltpu.sync_copy(w_ref.at[s, h], val_v.at[h])

        def _val(j, _):
            for q in range(8):
                sl = pl.ds(q * 16, 16)
                dr = plsc.load_gather(dinv_v, [row_h[j, sl]])
                dc = plsc.load_gather(dinv_v, [colp_h[j, sl]])
                val_v[h, j, sl] = dr * val_v[h, j, sl] * dc
            return 0
        lax.fori_loop(0, CPH, _val, 0)

    cv = coef_v[pl.ds(0, 16)]
    levels = [
        (y1_ref, xs_ref, None, (cv[1], cv[0], None)),
        (y2_ref, y1_ref, xs_ref, (cv[2], cv[3], cv[4])),
        (y3_ref, y2_ref, y1_ref, (cv[5], cv[6], cv[7])),
    ]
    for lvl, (out_ref, src_ref, p2_ref, (ta, tb, tc)) in enumerate(levels):
        def _zfill(r, _):
            for q in range(4):
                p2buf[r, pl.ds(q * 16, 16)] = zv
            return 0
        lax.fori_loop(0, CB, _zfill, 0)
        for r in range(RSUB):
            pltpu.sync_copy(p2buf, acc.at[pl.ds(rbase + r * CB, CB)])
        plsc.subcore_barrier()

        def _gs(j, buf, sem):
            return pltpu.async_copy(src_ref.at[colp_h.at[j]], buf, sem)

        def _gw(j, buf, sem):
            pltpu.make_async_copy(src_ref.at[colp_h.at[j]], buf, sem).wait()

        def _ss(j, buf, sem):
            pltpu.async_copy(buf, acc.at[row_h.at[j]], sem, add=True)

        def _sw(j, buf, sem):
            pltpu.make_async_copy(buf, acc.at[row_h.at[j]], sem).wait()

        def _scale(h, j, buf):
            def _sc(g, _):
                valv = val_v[h, j, pl.ds(g * 16, 16)]
                for e16 in range(16):
                    v = valv[e16]
                    e = g * 16 + e16
                    for q in range(4):
                        sl = pl.ds(q * 16, 16)
                        buf[e, sl] = buf[e, sl] * v
                return 0
            lax.fori_loop(0, CH // 16, _sc, 0)

        def _half(h, _):
            pltpu.sync_copy(row_ref.at[s, h], row_h)
            pltpu.sync_copy(col_ref.at[s, h], colp_h)

            def _cfix(j, _):
                for q in range(8):
                    sl = pl.ds(q * 16, 16)
                    colp_h[j, sl] = colp_h[j, sl] + off
                return 0
            lax.fori_loop(0, CPH, _cfix, 0)

            _gs(0, gbuf0, gsem0)
            _gs(1, gbuf1, gsem1)

            def _pair(t, _):
                j0 = 2 * t
                j1 = 2 * t + 1
                _gw(j0, gbuf0, gsem0)
                _scale(h, j0, gbuf0)
                _ss(j0, gbuf0, ssem0)
                _gw(j1, gbuf1, gsem1)
                _scale(h, j1, gbuf1)
                _ss(j1, gbuf1, ssem1)
                _sw(j0, gbuf0, ssem0)
                _gs(j0 + 2, gbuf0, gsem0)
                _sw(j1, gbuf1, ssem1)
                _gs(j1 + 2, gbuf1, gsem1)
                return 0
            lax.fori_loop(0, CPH // 2 - 1, _pair, 0)

            jt0 = CPH - 2
            jt1 = CPH - 1
            _gw(jt0, gbuf0, gsem0)
            _scale(h, jt0, gbuf0)
            _ss(jt0, gbuf0, ssem0)
            _gw(jt1, gbuf1, gsem1)
            _scale(h, jt1, gbuf1)
            _ss(jt1, gbuf1, ssem1)
            _sw(jt0, gbuf0, ssem0)
            _sw(jt1, gbuf1, ssem1)
            return 0
        lax.fori_loop(0, NH, _half, 0)
        plsc.subcore_barrier()

        for r in range(RSUB):
            base = rbase + r * CB
            hb = off + base
            ga = gbuf0.at[pl.ds(0, CB)]
            gp = gbuf1.at[pl.ds(0, CB)]
            pltpu.sync_copy(acc.at[pl.ds(base, CB)], ga)
            pltpu.sync_copy(src_ref.at[pl.ds(hb, CB)], gp)
            if p2_ref is not None:
                pltpu.sync_copy(p2_ref.at[pl.ds(hb, CB)], p2buf)

            def _cmb(i, _):
                for q in range(4):
                    sl = pl.ds(q * 16, 16)
                    a = gbuf0[i, sl]
                    p = gbuf1[i, sl]
                    if lvl == 0:
                        o = tb * p + ta * a
                    else:
                        o = ta * a - tb * p - tc * p2buf[i, sl]
                    gbuf0[i, sl] = o
                return 0
            lax.fori_loop(0, CB, _cmb, 0)
            pltpu.sync_copy(ga, out_ref.at[pl.ds(hb, CB)])


@jax.jit
def _poly_conv(x, edge_index, edge_attr, alphas):
    coefs = _coef_vector(alphas)

    zpad = jnp.zeros((NP - N, DH), jnp.float32)
    xs0 = jnp.concatenate([x[:, :DH], zpad, x[:, DH:], zpad], axis=0)

    row = edge_index[0].reshape(NS, EPT)
    col = edge_index[1].reshape(NS, EPT)
    w = edge_attr.reshape(NS, EPT)
    row = jnp.pad(row, ((0, 0), (0, EPT_P - EPT)), constant_values=N)
    col = jnp.pad(col, ((0, 0), (0, EPT_P - EPT)))
    w = jnp.pad(w, ((0, 0), (0, EPT_P - EPT)))
    row = row.reshape(NS, NH, CPH, CH)
    col = col.reshape(NS, NH, CPH, CH)
    w = w.reshape(NS, NH, CPH, CH)

    mesh = plsc.VectorSubcoreMesh(core_axis_name="c", subcore_axis_name="s")
    yshape = jax.ShapeDtypeStruct((NC * NP, DH), jnp.float32)
    run = pl.kernel(
        _sc_body,
        out_type=(yshape, yshape, yshape),
        mesh=mesh,
        compiler_params=pltpu.CompilerParams(
            needs_layout_passes=False, use_tc_tiling_on_sc=False),
        scratch_types=[
            pltpu.VMEM_SHARED((NP, DH), jnp.float32),
            pltpu.VMEM_SHARED((NP,), jnp.float32),
            pltpu.VMEM_SHARED((NP,), jnp.float32),
            pltpu.VMEM((CPH, CH), jnp.int32),
            pltpu.VMEM((CPH, CH), jnp.int32),
            pltpu.VMEM((NH, CPH, CH), jnp.float32),
            pltpu.VMEM((NP,), jnp.float32),
            pltpu.VMEM((CH, DH), jnp.float32),
            pltpu.VMEM((CH, DH), jnp.float32),
            pltpu.VMEM((CB, DH), jnp.float32),
            pltpu.VMEM((CH,), jnp.float32),
            pltpu.VMEM((RPT,), jnp.float32),
            pltpu.VMEM((16,), jnp.float32),
            pltpu.SemaphoreType.DMA,
            pltpu.SemaphoreType.DMA,
            pltpu.SemaphoreType.DMA,
            pltpu.SemaphoreType.DMA,
        ],
    )
    y1, y2, y3 = run(xs0, row, col, w, coefs)

    def halves(y):
        return jnp.concatenate([y[:N], y[NP:NP + N]], axis=1)
    return jnp.stack([x, halves(y1), halves(y2), halves(y3)], axis=1)


def kernel(x, edge_index, edge_attr, alphas):
    return _poly_conv(x, edge_index, edge_attr, alphas)
